# baseline (speedup 1.0000x reference)
import functools

import jax
import jax.numpy as jnp
from jax import lax
from jax.experimental import pallas as pl
from jax.experimental.pallas import tpu as pltpu
from jax.experimental.pallas import tpu_sc as plsc

L_EDGES = 16384
NC = 2
NS = 16
NW = NC * NS
EPW = L_EDGES // NW
CHUNK = 128
NCHUNK = EPW // CHUNK
LANES = 16
NVEC = EPW // LANES

_LN2 = 0.6931471805599453


def _ff(v):
    return jnp.full((LANES,), v, jnp.float32)


def _fi(v):
    return jnp.full((LANES,), v, jnp.int32)


def _vlog(x):
    xi = lax.bitcast_convert_type(x, jnp.int32)
    m = lax.bitcast_convert_type((xi & _fi(0x007FFFFF)) | _fi(0x3F800000),
                                 jnp.float32)
    e = (lax.shift_right_arithmetic(xi, _fi(23)) - _fi(127)).astype(jnp.float32)
    big = m > _ff(1.4142135)
    m = jnp.where(big, m * _ff(0.5), m)
    e = e + jnp.where(big, _ff(1.0), _ff(0.0))
    r = (m - _ff(1.0)) / (m + _ff(1.0))
    r2 = r * r
    t = ((_ff(1.0 / 7.0) * r2 + _ff(1.0 / 5.0)) * r2 + _ff(1.0 / 3.0)) * r2 + _ff(1.0)
    return e * _ff(_LN2) + _ff(2.0) * r * t


def _vsqrt(x):
    return jnp.exp(_ff(0.5) * _vlog(x))


def _sc_body(idx1, idx2, pidx1, pidx2, w, rs_loc, rs_scale, phf, phis_scale,
             consts, out, idx1_v, idx2_v, pidx1_v, pidx2_v, w_v,
             a1, a2, b1, b2, c1, c2,
             px1v, py1v, pz1v, px2v, py2v, pz2v, cv, ov, sem):
    wid = lax.axis_index("s") * NC + lax.axis_index("c")
    base = wid * EPW

    for j in range(NCHUNK):
        sl = pl.ds(base + j * CHUNK, CHUNK)
        pltpu.sync_copy(idx1.at[sl], idx1_v.at[j])
        pltpu.sync_copy(idx2.at[sl], idx2_v.at[j])
        for c in range(3):
            csl = pl.ds(c * L_EDGES + base + j * CHUNK, CHUNK)
            pltpu.sync_copy(pidx1.at[csl], pidx1_v.at[c, j])
            pltpu.sync_copy(pidx2.at[csl], pidx2_v.at[c, j])
    pltpu.sync_copy(w.at[pl.ds(base, EPW)], w_v)
    pltpu.sync_copy(consts, cv)

    copies = []
    for j in range(NCHUNK):
        i1 = idx1_v.at[j]
        i2 = idx2_v.at[j]
        dsl = pl.ds(j * CHUNK, CHUNK)
        copies.append(pltpu.async_copy(rs_loc.at[i1], a1.at[dsl], sem))
        copies.append(pltpu.async_copy(rs_loc.at[i2], a2.at[dsl], sem))
        copies.append(pltpu.async_copy(rs_scale.at[i1], b1.at[dsl], sem))
        copies.append(pltpu.async_copy(rs_scale.at[i2], b2.at[dsl], sem))
        copies.append(pltpu.async_copy(phis_scale.at[i1], c1.at[dsl], sem))
        copies.append(pltpu.async_copy(phis_scale.at[i2], c2.at[dsl], sem))
        for c, (d1, d2) in enumerate(((px1v, px2v), (py1v, py2v),
                                      (pz1v, pz2v))):
            copies.append(pltpu.async_copy(
                phf.at[pidx1_v.at[c, j]], d1.at[dsl], sem))
            copies.append(pltpu.async_copy(
                phf.at[pidx2_v.at[c, j]], d2.at[dsl], sem))
    for cp in copies:
        cp.wait()

    Rv = cv[0]
    itv = cv[1]
    av = cv[2]
    lnv = cv[3]
    ctv = cv[4]
    eps = _ff(1e-12)
    one = _ff(1.0)
    half = _ff(0.5)

    def chunk_body(k, _):
        sl = pl.ds(k * LANES, LANES)

        a1c = a1[sl]
        a2c = a2[sl]
        r1 = Rv / (one + jnp.exp(-a1c))
        r2 = Rv / (one + jnp.exp(-a2c))
        e1 = jnp.exp(r1)
        e2 = jnp.exp(r2)
        ch1 = half * (e1 + one / e1)
        sh1 = half * (e1 - one / e1)
        ch2 = half * (e2 + one / e2)
        sh2 = half * (e2 - one / e2)

        px1 = px1v[sl]
        py1 = py1v[sl]
        pz1 = pz1v[sl]
        px2 = px2v[sl]
        py2 = py2v[sl]
        pz2 = pz2v[sl]
        n1 = px1 * px1 + py1 * py1 + pz1 * pz1
        n2 = px2 * px2 + py2 * py2 + pz2 * pz2
        dot = px1 * px2 + py1 * py2 + pz1 * pz2
        cos = dot / ((_vsqrt(n1) + eps) * (_vsqrt(n2) + eps))
        cos = jnp.minimum(jnp.maximum(cos, -one), one)

        ch = jnp.maximum(ch1 * ch2 - sh1 * sh2 * cos, _ff(1.0 + 1e-7))
        d = _vlog(ch + _vsqrt(ch * ch - one))
        z = (d - Rv) * itv
        sp = _vlog(one + jnp.exp(-jnp.abs(z)))
        lim = _ff(-27.631021)
        lp = jnp.maximum(-(jnp.maximum(z, _ff(0.0)) + sp), lim)
        l1mp = jnp.maximum(-(jnp.maximum(-z, _ff(0.0)) + sp), lim)
        llt = jnp.where(w_v[sl] > _ff(0.0), lp, l1mp)

        g1 = jnp.exp(av * r1)
        g2 = jnp.exp(av * r2)
        logr1 = _vlog(av * half * (g1 - one / g1) + eps) - lnv
        logr2 = _vlog(av * half * (g2 - one / g2) + eps) - lnv

        s12 = jnp.exp(b1[sl]) + jnp.exp(c1[sl]) + jnp.exp(b2[sl]) + jnp.exp(c2[sl])

        ov[sl] = llt + logr1 + logr2 - _ff(1e-3) * s12 - ctv
        return 0

    lax.fori_loop(0, NVEC, chunk_body, 0)
    pltpu.sync_copy(ov, out.at[pl.ds(base, EPW)])


_sc_call = functools.partial(
    pl.kernel,
    out_type=jax.ShapeDtypeStruct((L_EDGES,), jnp.float32),
    mesh=plsc.VectorSubcoreMesh(core_axis_name="c", subcore_axis_name="s"),
    scratch_types=[
        pltpu.VMEM((NCHUNK, CHUNK), jnp.int32),
        pltpu.VMEM((NCHUNK, CHUNK), jnp.int32),
        pltpu.VMEM((3, NCHUNK, CHUNK), jnp.int32),
        pltpu.VMEM((3, NCHUNK, CHUNK), jnp.int32),
        pltpu.VMEM((EPW,), jnp.float32),
        pltpu.VMEM((EPW,), jnp.float32),
        pltpu.VMEM((EPW,), jnp.float32),
        pltpu.VMEM((EPW,), jnp.float32),
        pltpu.VMEM((EPW,), jnp.float32),
        pltpu.VMEM((EPW,), jnp.float32),
        pltpu.VMEM((EPW,), jnp.float32),
        pltpu.VMEM((EPW,), jnp.float32),
        pltpu.VMEM((EPW,), jnp.float32),
        pltpu.VMEM((EPW,), jnp.float32),
        pltpu.VMEM((EPW,), jnp.float32),
        pltpu.VMEM((EPW,), jnp.float32),
        pltpu.VMEM((EPW,), jnp.float32),
        pltpu.VMEM((8, LANES), jnp.float32),
        pltpu.VMEM((EPW,), jnp.float32),
        pltpu.SemaphoreType.DMA,
    ],
)(_sc_body)


def kernel(idx1, idx2, weights, rs_loc, rs_scale, phis_loc, phis_scale,
           R_loc, R_scale, T, alpha_loc, alpha_scale):
    f32 = jnp.float32
    eps = f32(1e-12)
    R = jnp.exp(R_loc)
    T_x = jnp.exp(T)
    T_s = T_x[0] / (T_x[0] + T_x[1])
    alpha = jnp.exp(alpha_loc)
    inv_t = f32(1.0) / (f32(2.0) * T_s + eps)
    log_norm = jnp.log(jnp.cosh(alpha * R) - f32(1.0) + eps)
    kl_glob = (f32(0.5) * (R_loc ** 2 + jnp.exp(R_scale) ** 2)
               + f32(0.5) * (alpha_loc ** 2 + jnp.exp(alpha_scale) ** 2))
    cterm = kl_glob / f32(L_EDGES)
    consts = jnp.stack([R, inv_t, alpha, log_norm, cterm,
                        f32(0.0), f32(0.0), f32(0.0)]).astype(f32)
    consts16 = jnp.broadcast_to(consts[:, None], (8, LANES))
    i1 = idx1.astype(jnp.int32)
    i2 = idx2.astype(jnp.int32)
    comp = jnp.arange(3, dtype=jnp.int32)[:, None]
    pidx1 = (3 * i1[None, :] + comp).reshape(-1)
    pidx2 = (3 * i2[None, :] + comp).reshape(-1)
    one_rt = jnp.exp(R_loc * f32(0.0))
    phf = (phis_loc.astype(f32) * one_rt).reshape(-1)
    return _sc_call(i1, i2, pidx1, pidx2, weights.astype(f32),
                    rs_loc.astype(f32), rs_scale.astype(f32), phf,
                    phis_scale.astype(f32), consts16)


# speedup vs baseline: 37.0522x; 37.0522x over previous
import functools

import jax
import jax.numpy as jnp
from jax import lax
from jax.experimental import pallas as pl
from jax.experimental.pallas import tpu as pltpu
from jax.experimental.pallas import tpu_sc as plsc

L_EDGES = 16384
NC = 2
NS = 16
NW = NC * NS
EPW = L_EDGES // NW
CHUNK = 128
NCHUNK = EPW // CHUNK
LANES = 16
VPC = CHUNK // LANES

_LN2 = 0.6931471805599453


def _ff(v):
    return jnp.full((LANES,), v, jnp.float32)


def _fi(v):
    return jnp.full((LANES,), v, jnp.int32)


def _vlog(x):
    xi = lax.bitcast_convert_type(x, jnp.int32)
    m = lax.bitcast_convert_type((xi & _fi(0x007FFFFF)) | _fi(0x3F800000),
                                 jnp.float32)
    e = (lax.shift_right_arithmetic(xi, _fi(23)) - _fi(127)).astype(jnp.float32)
    big = m > _ff(1.4142135)
    m = jnp.where(big, m * _ff(0.5), m)
    e = e + jnp.where(big, _ff(1.0), _ff(0.0))
    r = (m - _ff(1.0)) / (m + _ff(1.0))
    r2 = r * r
    t = ((_ff(1.0 / 7.0) * r2 + _ff(1.0 / 5.0)) * r2 + _ff(1.0 / 3.0)) * r2 + _ff(1.0)
    return e * _ff(_LN2) + _ff(2.0) * r * t


def _vsqrt(x):
    return jnp.exp(_ff(0.5) * _vlog(x))


def _sc_a(idx1, idx2, rs_loc, rs_scale, phis_scale, consts,
          cc_o, ss_o, rest_o, idx1_v, idx2_v, a1, a2, b1, b2, c1, c2,
          cv, cc_v, ss_v, rest_v, sem):
    wid = lax.axis_index("s") * NC + lax.axis_index("c")
    base = wid * EPW

    for j in range(NCHUNK):
        sl = pl.ds(base + j * CHUNK, CHUNK)
        pltpu.sync_copy(idx1.at[sl], idx1_v.at[j])
        pltpu.sync_copy(idx2.at[sl], idx2_v.at[j])
    pltpu.sync_copy(consts, cv)

    chunk_copies = []
    for j in range(NCHUNK):
        i1 = idx1_v.at[j]
        i2 = idx2_v.at[j]
        dsl = pl.ds(j * CHUNK, CHUNK)
        chunk_copies.append([
            pltpu.async_copy(rs_loc.at[i1], a1.at[dsl], sem),
            pltpu.async_copy(rs_loc.at[i2], a2.at[dsl], sem),
            pltpu.async_copy(rs_scale.at[i1], b1.at[dsl], sem),
            pltpu.async_copy(rs_scale.at[i2], b2.at[dsl], sem),
            pltpu.async_copy(phis_scale.at[i1], c1.at[dsl], sem),
            pltpu.async_copy(phis_scale.at[i2], c2.at[dsl], sem),
        ])

    Rv = cv[0]
    av = cv[2]
    lnv = cv[3]
    ctv = cv[4]
    eps = _ff(1e-12)
    one = _ff(1.0)
    half = _ff(0.5)

    def vec_body(k, _):
        sl = pl.ds(k * LANES, LANES)
        r1 = Rv / (one + jnp.exp(-a1[sl]))
        r2 = Rv / (one + jnp.exp(-a2[sl]))
        e1 = jnp.exp(r1)
        e2 = jnp.exp(r2)
        ch1 = half * (e1 + one / e1)
        sh1 = half * (e1 - one / e1)
        ch2 = half * (e2 + one / e2)
        sh2 = half * (e2 - one / e2)
        g1 = jnp.exp(av * r1)
        g2 = jnp.exp(av * r2)
        logr1 = _vlog(av * half * (g1 - one / g1) + eps) - lnv
        logr2 = _vlog(av * half * (g2 - one / g2) + eps) - lnv
        s12 = jnp.exp(b1[sl]) + jnp.exp(c1[sl]) + jnp.exp(b2[sl]) + jnp.exp(c2[sl])
        cc_v[sl] = ch1 * ch2
        ss_v[sl] = sh1 * sh2
        rest_v[sl] = logr1 + logr2 - _ff(1e-3) * s12 - ctv
        return 0

    for j in range(NCHUNK):
        for cp in chunk_copies[j]:
            cp.wait()
        lax.fori_loop(j * VPC, (j + 1) * VPC, vec_body, 0)

    osl = pl.ds(base, EPW)
    pltpu.sync_copy(cc_v, cc_o.at[osl])
    pltpu.sync_copy(ss_v, ss_o.at[osl])
    pltpu.sync_copy(rest_v, rest_o.at[osl])


def _sc_b(idx1, idx2, w, phx, phy, phz, cc, ss, rest, consts, out,
          idx1_v, idx2_v, w_v, cc_v, ss_v, rest_v,
          px1v, py1v, pz1v, px2v, py2v, pz2v, cv, ov, sem):
    wid = lax.axis_index("s") * NC + lax.axis_index("c")
    base = wid * EPW

    for j in range(NCHUNK):
        sl = pl.ds(base + j * CHUNK, CHUNK)
        pltpu.sync_copy(idx1.at[sl], idx1_v.at[j])
        pltpu.sync_copy(idx2.at[sl], idx2_v.at[j])
    bsl = pl.ds(base, EPW)
    pltpu.sync_copy(w.at[bsl], w_v)
    pltpu.sync_copy(cc.at[bsl], cc_v)
    pltpu.sync_copy(ss.at[bsl], ss_v)
    pltpu.sync_copy(rest.at[bsl], rest_v)
    pltpu.sync_copy(consts, cv)

    chunk_copies = []
    for j in range(NCHUNK):
        i1 = idx1_v.at[j]
        i2 = idx2_v.at[j]
        dsl = pl.ds(j * CHUNK, CHUNK)
        chunk_copies.append([
            pltpu.async_copy(phx.at[i1], px1v.at[dsl], sem),
            pltpu.async_copy(phx.at[i2], px2v.at[dsl], sem),
            pltpu.async_copy(phy.at[i1], py1v.at[dsl], sem),
            pltpu.async_copy(phy.at[i2], py2v.at[dsl], sem),
            pltpu.async_copy(phz.at[i1], pz1v.at[dsl], sem),
            pltpu.async_copy(phz.at[i2], pz2v.at[dsl], sem),
        ])

    Rv = cv[0]
    itv = cv[1]
    eps = _ff(1e-12)
    one = _ff(1.0)

    def vec_body(k, _):
        sl = pl.ds(k * LANES, LANES)
        px1 = px1v[sl]
        py1 = py1v[sl]
        pz1 = pz1v[sl]
        px2 = px2v[sl]
        py2 = py2v[sl]
        pz2 = pz2v[sl]
        n1 = px1 * px1 + py1 * py1 + pz1 * pz1
        n2 = px2 * px2 + py2 * py2 + pz2 * pz2
        dot = px1 * px2 + py1 * py2 + pz1 * pz2
        cos = dot / ((_vsqrt(n1) + eps) * (_vsqrt(n2) + eps))
        cos = jnp.minimum(jnp.maximum(cos, -one), one)
        ch = jnp.maximum(cc_v[sl] - ss_v[sl] * cos, _ff(1.0 + 1e-7))
        d = _vlog(ch + _vsqrt(ch * ch - one))
        z = (d - Rv) * itv
        sp = _vlog(one + jnp.exp(-jnp.abs(z)))
        lim = _ff(-27.631021)
        lp = jnp.maximum(-(jnp.maximum(z, _ff(0.0)) + sp), lim)
        l1mp = jnp.maximum(-(jnp.maximum(-z, _ff(0.0)) + sp), lim)
        llt = jnp.where(w_v[sl] > _ff(0.0), lp, l1mp)
        ov[sl] = llt + rest_v[sl]
        return 0

    for j in range(NCHUNK):
        for cp in chunk_copies[j]:
            cp.wait()
        lax.fori_loop(j * VPC, (j + 1) * VPC, vec_body, 0)

    pltpu.sync_copy(ov, out.at[pl.ds(base, EPW)])


_mesh = plsc.VectorSubcoreMesh(core_axis_name="c", subcore_axis_name="s")

_F32L = jax.ShapeDtypeStruct((L_EDGES,), jnp.float32)

_sc_call_a = functools.partial(
    pl.kernel,
    out_type=(_F32L, _F32L, _F32L),
    mesh=_mesh,
    scratch_types=[
        pltpu.VMEM((NCHUNK, CHUNK), jnp.int32),
        pltpu.VMEM((NCHUNK, CHUNK), jnp.int32),
        pltpu.VMEM((EPW,), jnp.float32),
        pltpu.VMEM((EPW,), jnp.float32),
        pltpu.VMEM((EPW,), jnp.float32),
        pltpu.VMEM((EPW,), jnp.float32),
        pltpu.VMEM((EPW,), jnp.float32),
        pltpu.VMEM((EPW,), jnp.float32),
        pltpu.VMEM((8, LANES), jnp.float32),
        pltpu.VMEM((EPW,), jnp.float32),
        pltpu.VMEM((EPW,), jnp.float32),
        pltpu.VMEM((EPW,), jnp.float32),
        pltpu.SemaphoreType.DMA,
    ],
)(_sc_a)

_sc_call_b = functools.partial(
    pl.kernel,
    out_type=_F32L,
    mesh=_mesh,
    scratch_types=[
        pltpu.VMEM((NCHUNK, CHUNK), jnp.int32),
        pltpu.VMEM((NCHUNK, CHUNK), jnp.int32),
        pltpu.VMEM((EPW,), jnp.float32),
        pltpu.VMEM((EPW,), jnp.float32),
        pltpu.VMEM((EPW,), jnp.float32),
        pltpu.VMEM((EPW,), jnp.float32),
        pltpu.VMEM((EPW,), jnp.float32),
        pltpu.VMEM((EPW,), jnp.float32),
        pltpu.VMEM((EPW,), jnp.float32),
        pltpu.VMEM((EPW,), jnp.float32),
        pltpu.VMEM((EPW,), jnp.float32),
        pltpu.VMEM((EPW,), jnp.float32),
        pltpu.VMEM((8, LANES), jnp.float32),
        pltpu.VMEM((EPW,), jnp.float32),
        pltpu.SemaphoreType.DMA,
    ],
)(_sc_b)


def kernel(idx1, idx2, weights, rs_loc, rs_scale, phis_loc, phis_scale,
           R_loc, R_scale, T, alpha_loc, alpha_scale):
    f32 = jnp.float32
    eps = f32(1e-12)
    R = jnp.exp(R_loc)
    T_x = jnp.exp(T)
    T_s = T_x[0] / (T_x[0] + T_x[1])
    alpha = jnp.exp(alpha_loc)
    inv_t = f32(1.0) / (f32(2.0) * T_s + eps)
    log_norm = jnp.log(jnp.cosh(alpha * R) - f32(1.0) + eps)
    kl_glob = (f32(0.5) * (R_loc ** 2 + jnp.exp(R_scale) ** 2)
               + f32(0.5) * (alpha_loc ** 2 + jnp.exp(alpha_scale) ** 2))
    cterm = kl_glob / f32(L_EDGES)
    consts = jnp.stack([R, inv_t, alpha, log_norm, cterm,
                        f32(0.0), f32(0.0), f32(0.0)]).astype(f32)
    consts16 = jnp.broadcast_to(consts[:, None], (8, LANES))
    i1 = idx1.astype(jnp.int32)
    i2 = idx2.astype(jnp.int32)
    ph = phis_loc.astype(f32)
    cc, ss, rest = _sc_call_a(i1, i2, rs_loc.astype(f32),
                              rs_scale.astype(f32), phis_scale.astype(f32),
                              consts16)
    return _sc_call_b(i1, i2, weights.astype(f32), ph[:, 0], ph[:, 1],
                      ph[:, 2], cc, ss, rest, consts16)


# speedup vs baseline: 40.1106x; 1.0825x over previous
import functools

import jax
import jax.numpy as jnp
from jax import lax
from jax.experimental import pallas as pl
from jax.experimental.pallas import tpu as pltpu
from jax.experimental.pallas import tpu_sc as plsc

L_EDGES = 16384
NC = 2
NS = 16
NW = NC * NS
EPW = L_EDGES // NW
CHUNK = 128
NCHUNK = EPW // CHUNK
LANES = 16
VPC = CHUNK // LANES

_LN2 = 0.6931471805599453


def _ff(v):
    return jnp.full((LANES,), v, jnp.float32)


def _fi(v):
    return jnp.full((LANES,), v, jnp.int32)


def _vlog(x):
    xi = lax.bitcast_convert_type(x, jnp.int32)
    m = lax.bitcast_convert_type((xi & _fi(0x007FFFFF)) | _fi(0x3F800000),
                                 jnp.float32)
    e = (lax.shift_right_arithmetic(xi, _fi(23)) - _fi(127)).astype(jnp.float32)
    big = m > _ff(1.4142135)
    m = jnp.where(big, m * _ff(0.5), m)
    e = e + jnp.where(big, _ff(1.0), _ff(0.0))
    r = (m - _ff(1.0)) / (m + _ff(1.0))
    r2 = r * r
    t = ((_ff(1.0 / 7.0) * r2 + _ff(1.0 / 5.0)) * r2 + _ff(1.0 / 3.0)) * r2 + _ff(1.0)
    return e * _ff(_LN2) + _ff(2.0) * r * t


def _vsqrt(x):
    return jnp.exp(_ff(0.5) * _vlog(x))


def _sc_a(idx1, idx2, rs_loc, rs_scale, phis_scale, consts,
          cc_o, ss_o, rest_o, idx1_v, idx2_v, a1, a2, b1, b2, c1, c2,
          cv, cc_v, ss_v, rest_v, sem):
    wid = lax.axis_index("s") * NC + lax.axis_index("c")
    base = wid * EPW

    for j in range(NCHUNK):
        sl = pl.ds(base + j * CHUNK, CHUNK)
        pltpu.sync_copy(idx1.at[sl], idx1_v.at[j])
        pltpu.sync_copy(idx2.at[sl], idx2_v.at[j])
    pltpu.sync_copy(consts, cv)

    chunk_copies = []
    for j in range(NCHUNK):
        i1 = idx1_v.at[j]
        i2 = idx2_v.at[j]
        dsl = pl.ds(j * CHUNK, CHUNK)
        chunk_copies.append([
            pltpu.async_copy(rs_loc.at[i1], a1.at[dsl], sem),
            pltpu.async_copy(rs_loc.at[i2], a2.at[dsl], sem),
            pltpu.async_copy(rs_scale.at[i1], b1.at[dsl], sem),
            pltpu.async_copy(rs_scale.at[i2], b2.at[dsl], sem),
            pltpu.async_copy(phis_scale.at[i1], c1.at[dsl], sem),
            pltpu.async_copy(phis_scale.at[i2], c2.at[dsl], sem),
        ])

    Rv = cv[0]
    av = cv[2]
    lnv = cv[3]
    ctv = cv[4]
    eps = _ff(1e-12)
    one = _ff(1.0)
    half = _ff(0.5)

    def vec_body(k, _):
        sl = pl.ds(k * LANES, LANES)
        r1 = Rv / (one + jnp.exp(-a1[sl]))
        r2 = Rv / (one + jnp.exp(-a2[sl]))
        e1 = jnp.exp(r1)
        e2 = jnp.exp(r2)
        ch1 = half * (e1 + one / e1)
        sh1 = half * (e1 - one / e1)
        ch2 = half * (e2 + one / e2)
        sh2 = half * (e2 - one / e2)
        g1 = jnp.exp(av * r1)
        g2 = jnp.exp(av * r2)
        logr1 = _vlog(av * half * (g1 - one / g1) + eps) - lnv
        logr2 = _vlog(av * half * (g2 - one / g2) + eps) - lnv
        s12 = jnp.exp(b1[sl]) + jnp.exp(c1[sl]) + jnp.exp(b2[sl]) + jnp.exp(c2[sl])
        cc_v[sl] = ch1 * ch2
        ss_v[sl] = sh1 * sh2
        rest_v[sl] = logr1 + logr2 - _ff(1e-3) * s12 - ctv
        return 0

    for j in range(NCHUNK):
        for cp in chunk_copies[j]:
            cp.wait()
        lax.fori_loop(j * VPC, (j + 1) * VPC, vec_body, 0)

    osl = pl.ds(base, EPW)
    pltpu.sync_copy(cc_v, cc_o.at[osl])
    pltpu.sync_copy(ss_v, ss_o.at[osl])
    pltpu.sync_copy(rest_v, rest_o.at[osl])


def _sc_b(idx1, idx2, w, phx, phy, phz, cc, ss, rest, consts, out,
          idx1_v, idx2_v, w_v, cc_v, ss_v, rest_v,
          px1v, py1v, pz1v, px2v, py2v, pz2v, cv, ov, sem, sem2):
    wid = lax.axis_index("s") * NC + lax.axis_index("c")
    base = wid * EPW

    idx_cps = []
    for j in range(NCHUNK):
        sl = pl.ds(base + j * CHUNK, CHUNK)
        idx_cps.append(pltpu.async_copy(idx1.at[sl], idx1_v.at[j], sem2))
        idx_cps.append(pltpu.async_copy(idx2.at[sl], idx2_v.at[j], sem2))
    bsl = pl.ds(base, EPW)
    stage_cps = [
        pltpu.async_copy(w.at[bsl], w_v, sem2),
        pltpu.async_copy(cc.at[bsl], cc_v, sem2),
        pltpu.async_copy(ss.at[bsl], ss_v, sem2),
        pltpu.async_copy(rest.at[bsl], rest_v, sem2),
        pltpu.async_copy(consts, cv, sem2),
    ]
    for cp in idx_cps:
        cp.wait()

    chunk_copies = []
    for j in range(NCHUNK):
        i1 = idx1_v.at[j]
        i2 = idx2_v.at[j]
        dsl = pl.ds(j * CHUNK, CHUNK)
        chunk_copies.append([
            pltpu.async_copy(phx.at[i1], px1v.at[dsl], sem),
            pltpu.async_copy(phx.at[i2], px2v.at[dsl], sem),
            pltpu.async_copy(phy.at[i1], py1v.at[dsl], sem),
            pltpu.async_copy(phy.at[i2], py2v.at[dsl], sem),
            pltpu.async_copy(phz.at[i1], pz1v.at[dsl], sem),
            pltpu.async_copy(phz.at[i2], pz2v.at[dsl], sem),
        ])

    for cp in stage_cps:
        cp.wait()
    Rv = cv[0]
    itv = cv[1]
    one = _ff(1.0)
    half = _ff(0.5)

    def vec_body(k, _):
        sl = pl.ds(k * LANES, LANES)
        px1 = px1v[sl]
        py1 = py1v[sl]
        pz1 = pz1v[sl]
        px2 = px2v[sl]
        py2 = py2v[sl]
        pz2 = pz2v[sl]
        n1 = px1 * px1 + py1 * py1 + pz1 * pz1
        n2 = px2 * px2 + py2 * py2 + pz2 * pz2
        dot = px1 * px2 + py1 * py2 + pz1 * pz2
        cos = dot * jnp.exp(-half * _vlog(n1 * n2))
        cos = jnp.minimum(jnp.maximum(cos, -one), one)
        ch = jnp.maximum(cc_v[sl] - ss_v[sl] * cos, _ff(1.0 + 1e-7))
        d = _vlog(ch + _vsqrt(ch * ch - one))
        z = (d - Rv) * itv
        sp = _vlog(one + jnp.exp(-jnp.abs(z)))
        lim = _ff(-27.631021)
        lp = jnp.maximum(-(jnp.maximum(z, _ff(0.0)) + sp), lim)
        l1mp = jnp.maximum(-(jnp.maximum(-z, _ff(0.0)) + sp), lim)
        llt = jnp.where(w_v[sl] > _ff(0.0), lp, l1mp)
        ov[sl] = llt + rest_v[sl]
        return 0

    for j in range(NCHUNK):
        for cp in chunk_copies[j]:
            cp.wait()
        lax.fori_loop(j * VPC, (j + 1) * VPC, vec_body, 0)

    pltpu.sync_copy(ov, out.at[pl.ds(base, EPW)])


_mesh = plsc.VectorSubcoreMesh(core_axis_name="c", subcore_axis_name="s")

_F32L = jax.ShapeDtypeStruct((L_EDGES,), jnp.float32)

_sc_call_a = functools.partial(
    pl.kernel,
    out_type=(_F32L, _F32L, _F32L),
    mesh=_mesh,
    scratch_types=[
        pltpu.VMEM((NCHUNK, CHUNK), jnp.int32),
        pltpu.VMEM((NCHUNK, CHUNK), jnp.int32),
        pltpu.VMEM((EPW,), jnp.float32),
        pltpu.VMEM((EPW,), jnp.float32),
        pltpu.VMEM((EPW,), jnp.float32),
        pltpu.VMEM((EPW,), jnp.float32),
        pltpu.VMEM((EPW,), jnp.float32),
        pltpu.VMEM((EPW,), jnp.float32),
        pltpu.VMEM((8, LANES), jnp.float32),
        pltpu.VMEM((EPW,), jnp.float32),
        pltpu.VMEM((EPW,), jnp.float32),
        pltpu.VMEM((EPW,), jnp.float32),
        pltpu.SemaphoreType.DMA,
    ],
)(_sc_a)

_sc_call_b = functools.partial(
    pl.kernel,
    out_type=_F32L,
    mesh=_mesh,
    scratch_types=[
        pltpu.VMEM((NCHUNK, CHUNK), jnp.int32),
        pltpu.VMEM((NCHUNK, CHUNK), jnp.int32),
        pltpu.VMEM((EPW,), jnp.float32),
        pltpu.VMEM((EPW,), jnp.float32),
        pltpu.VMEM((EPW,), jnp.float32),
        pltpu.VMEM((EPW,), jnp.float32),
        pltpu.VMEM((EPW,), jnp.float32),
        pltpu.VMEM((EPW,), jnp.float32),
        pltpu.VMEM((EPW,), jnp.float32),
        pltpu.VMEM((EPW,), jnp.float32),
        pltpu.VMEM((EPW,), jnp.float32),
        pltpu.VMEM((EPW,), jnp.float32),
        pltpu.VMEM((8, LANES), jnp.float32),
        pltpu.VMEM((EPW,), jnp.float32),
        pltpu.SemaphoreType.DMA,
        pltpu.SemaphoreType.DMA,
    ],
)(_sc_b)


def kernel(idx1, idx2, weights, rs_loc, rs_scale, phis_loc, phis_scale,
           R_loc, R_scale, T, alpha_loc, alpha_scale):
    f32 = jnp.float32
    eps = f32(1e-12)
    R = jnp.exp(R_loc)
    T_x = jnp.exp(T)
    T_s = T_x[0] / (T_x[0] + T_x[1])
    alpha = jnp.exp(alpha_loc)
    inv_t = f32(1.0) / (f32(2.0) * T_s + eps)
    log_norm = jnp.log(jnp.cosh(alpha * R) - f32(1.0) + eps)
    kl_glob = (f32(0.5) * (R_loc ** 2 + jnp.exp(R_scale) ** 2)
               + f32(0.5) * (alpha_loc ** 2 + jnp.exp(alpha_scale) ** 2))
    cterm = kl_glob / f32(L_EDGES)
    consts = jnp.stack([R, inv_t, alpha, log_norm, cterm,
                        f32(0.0), f32(0.0), f32(0.0)]).astype(f32)
    consts16 = jnp.broadcast_to(consts[:, None], (8, LANES))
    i1 = idx1.astype(jnp.int32)
    i2 = idx2.astype(jnp.int32)
    ph = phis_loc.astype(f32)
    cc, ss, rest = _sc_call_a(i1, i2, rs_loc.astype(f32),
                              rs_scale.astype(f32), phis_scale.astype(f32),
                              consts16)
    return _sc_call_b(i1, i2, weights.astype(f32), ph[:, 0], ph[:, 1],
                      ph[:, 2], cc, ss, rest, consts16)


# speedup vs baseline: 40.8385x; 1.0181x over previous
import functools

import jax
import jax.numpy as jnp
from jax import lax
from jax.experimental import pallas as pl
from jax.experimental.pallas import tpu as pltpu
from jax.experimental.pallas import tpu_sc as plsc

L_EDGES = 16384
NC = 2
NS = 16
NW = NC * NS
EPW = L_EDGES // NW
CHUNK = 128
NCHUNK = EPW // CHUNK
LANES = 16
VPC = CHUNK // LANES

_LN2 = 0.6931471805599453


def _ff(v):
    return jnp.full((LANES,), v, jnp.float32)


def _fi(v):
    return jnp.full((LANES,), v, jnp.int32)


def _vlog(x):
    xi = lax.bitcast_convert_type(x, jnp.int32)
    m = lax.bitcast_convert_type((xi & _fi(0x007FFFFF)) | _fi(0x3F800000),
                                 jnp.float32)
    e = (lax.shift_right_arithmetic(xi, _fi(23)) - _fi(127)).astype(jnp.float32)
    big = m > _ff(1.4142135)
    m = jnp.where(big, m * _ff(0.5), m)
    e = e + jnp.where(big, _ff(1.0), _ff(0.0))
    r = (m - _ff(1.0)) / (m + _ff(1.0))
    r2 = r * r
    t = ((_ff(1.0 / 7.0) * r2 + _ff(1.0 / 5.0)) * r2 + _ff(1.0 / 3.0)) * r2 + _ff(1.0)
    return e * _ff(_LN2) + _ff(2.0) * r * t


def _vsqrt(x):
    return jnp.exp(_ff(0.5) * _vlog(x))


def _sc_a(idx1, idx2, rs_loc, rs_scale, phis_scale, consts,
          cc_o, ss_o, rest_o, idx1_v, idx2_v, a1, a2, b1, b2, c1, c2,
          cv, cc_v, ss_v, rest_v, sem):
    wid = lax.axis_index("s") * NC + lax.axis_index("c")
    base = wid * EPW

    for j in range(NCHUNK):
        sl = pl.ds(base + j * CHUNK, CHUNK)
        pltpu.sync_copy(idx1.at[sl], idx1_v.at[j])
        pltpu.sync_copy(idx2.at[sl], idx2_v.at[j])
    pltpu.sync_copy(consts, cv)

    chunk_copies = []
    for j in range(NCHUNK):
        i1 = idx1_v.at[j]
        i2 = idx2_v.at[j]
        dsl = pl.ds(j * CHUNK, CHUNK)
        chunk_copies.append([
            pltpu.async_copy(rs_loc.at[i1], a1.at[dsl], sem),
            pltpu.async_copy(rs_loc.at[i2], a2.at[dsl], sem),
            pltpu.async_copy(rs_scale.at[i1], b1.at[dsl], sem),
            pltpu.async_copy(rs_scale.at[i2], b2.at[dsl], sem),
            pltpu.async_copy(phis_scale.at[i1], c1.at[dsl], sem),
            pltpu.async_copy(phis_scale.at[i2], c2.at[dsl], sem),
        ])

    eps = _ff(1e-12)
    one = _ff(1.0)
    half = _ff(0.5)
    Rv = jnp.exp(cv[0])
    av = jnp.exp(cv[4])
    ear = jnp.exp(av * Rv)
    lnv = _vlog(half * (ear + one / ear) - one + eps)
    er_s = jnp.exp(cv[1])
    ea_s = jnp.exp(cv[5])
    kl = half * (cv[0] * cv[0] + er_s * er_s) + half * (cv[4] * cv[4] + ea_s * ea_s)
    ctv = kl * _ff(1.0 / L_EDGES)

    def vec_body(k, _):
        sl = pl.ds(k * LANES, LANES)
        r1 = Rv / (one + jnp.exp(-a1[sl]))
        r2 = Rv / (one + jnp.exp(-a2[sl]))
        e1 = jnp.exp(r1)
        e2 = jnp.exp(r2)
        ch1 = half * (e1 + one / e1)
        sh1 = half * (e1 - one / e1)
        ch2 = half * (e2 + one / e2)
        sh2 = half * (e2 - one / e2)
        g1 = jnp.exp(av * r1)
        g2 = jnp.exp(av * r2)
        logr1 = _vlog(av * half * (g1 - one / g1) + eps) - lnv
        logr2 = _vlog(av * half * (g2 - one / g2) + eps) - lnv
        s12 = jnp.exp(b1[sl]) + jnp.exp(c1[sl]) + jnp.exp(b2[sl]) + jnp.exp(c2[sl])
        cc_v[sl] = ch1 * ch2
        ss_v[sl] = sh1 * sh2
        rest_v[sl] = logr1 + logr2 - _ff(1e-3) * s12 - ctv
        return 0

    for j in range(NCHUNK):
        for cp in chunk_copies[j]:
            cp.wait()
        lax.fori_loop(j * VPC, (j + 1) * VPC, vec_body, 0)

    osl = pl.ds(base, EPW)
    pltpu.sync_copy(cc_v, cc_o.at[osl])
    pltpu.sync_copy(ss_v, ss_o.at[osl])
    pltpu.sync_copy(rest_v, rest_o.at[osl])


def _sc_b(idx1, idx2, w, phx, phy, phz, cc, ss, rest, consts, out,
          idx1_v, idx2_v, w_v, cc_v, ss_v, rest_v,
          px1v, py1v, pz1v, px2v, py2v, pz2v, cv, ov, sem, sem2):
    wid = lax.axis_index("s") * NC + lax.axis_index("c")
    base = wid * EPW

    idx_cps = []
    for j in range(NCHUNK):
        sl = pl.ds(base + j * CHUNK, CHUNK)
        idx_cps.append(pltpu.async_copy(idx1.at[sl], idx1_v.at[j], sem2))
        idx_cps.append(pltpu.async_copy(idx2.at[sl], idx2_v.at[j], sem2))
    bsl = pl.ds(base, EPW)
    stage_cps = [
        pltpu.async_copy(w.at[bsl], w_v, sem2),
        pltpu.async_copy(cc.at[bsl], cc_v, sem2),
        pltpu.async_copy(ss.at[bsl], ss_v, sem2),
        pltpu.async_copy(rest.at[bsl], rest_v, sem2),
        pltpu.async_copy(consts, cv, sem2),
    ]
    for cp in idx_cps:
        cp.wait()

    chunk_copies = []
    for j in range(NCHUNK):
        i1 = idx1_v.at[j]
        i2 = idx2_v.at[j]
        dsl = pl.ds(j * CHUNK, CHUNK)
        chunk_copies.append([
            pltpu.async_copy(phx.at[i1], px1v.at[dsl], sem),
            pltpu.async_copy(phx.at[i2], px2v.at[dsl], sem),
            pltpu.async_copy(phy.at[i1], py1v.at[dsl], sem),
            pltpu.async_copy(phy.at[i2], py2v.at[dsl], sem),
            pltpu.async_copy(phz.at[i1], pz1v.at[dsl], sem),
            pltpu.async_copy(phz.at[i2], pz2v.at[dsl], sem),
        ])

    for cp in stage_cps:
        cp.wait()
    one = _ff(1.0)
    half = _ff(0.5)
    eps = _ff(1e-12)
    Rv = jnp.exp(cv[0])
    tx0 = jnp.exp(cv[2])
    tx1 = jnp.exp(cv[3])
    itv = one / (_ff(2.0) * (tx0 / (tx0 + tx1)) + eps)

    def vec_body(k, _):
        sl = pl.ds(k * LANES, LANES)
        px1 = px1v[sl]
        py1 = py1v[sl]
        pz1 = pz1v[sl]
        px2 = px2v[sl]
        py2 = py2v[sl]
        pz2 = pz2v[sl]
        n1 = px1 * px1 + py1 * py1 + pz1 * pz1
        n2 = px2 * px2 + py2 * py2 + pz2 * pz2
        dot = px1 * px2 + py1 * py2 + pz1 * pz2
        cos = dot * jnp.exp(-half * _vlog(n1 * n2))
        cos = jnp.minimum(jnp.maximum(cos, -one), one)
        ch = jnp.maximum(cc_v[sl] - ss_v[sl] * cos, _ff(1.0 + 1e-7))
        d = _vlog(ch + _vsqrt(ch * ch - one))
        z = (d - Rv) * itv
        sp = _vlog(one + jnp.exp(-jnp.abs(z)))
        lim = _ff(-27.631021)
        lp = jnp.maximum(-(jnp.maximum(z, _ff(0.0)) + sp), lim)
        l1mp = jnp.maximum(-(jnp.maximum(-z, _ff(0.0)) + sp), lim)
        llt = jnp.where(w_v[sl] > _ff(0.0), lp, l1mp)
        ov[sl] = llt + rest_v[sl]

    for j in range(NCHUNK):
        for cp in chunk_copies[j]:
            cp.wait()
        plsc.parallel_loop(j * VPC, (j + 1) * VPC, 1, unroll=2)(
            lambda k: vec_body(k, None))

    pltpu.sync_copy(ov, out.at[pl.ds(base, EPW)])


_mesh = plsc.VectorSubcoreMesh(core_axis_name="c", subcore_axis_name="s")

_F32L = jax.ShapeDtypeStruct((L_EDGES,), jnp.float32)

_sc_call_a = functools.partial(
    pl.kernel,
    out_type=(_F32L, _F32L, _F32L),
    mesh=_mesh,
    scratch_types=[
        pltpu.VMEM((NCHUNK, CHUNK), jnp.int32),
        pltpu.VMEM((NCHUNK, CHUNK), jnp.int32),
        pltpu.VMEM((EPW,), jnp.float32),
        pltpu.VMEM((EPW,), jnp.float32),
        pltpu.VMEM((EPW,), jnp.float32),
        pltpu.VMEM((EPW,), jnp.float32),
        pltpu.VMEM((EPW,), jnp.float32),
        pltpu.VMEM((EPW,), jnp.float32),
        pltpu.VMEM((8, LANES), jnp.float32),
        pltpu.VMEM((EPW,), jnp.float32),
        pltpu.VMEM((EPW,), jnp.float32),
        pltpu.VMEM((EPW,), jnp.float32),
        pltpu.SemaphoreType.DMA,
    ],
)(_sc_a)

_sc_call_b = functools.partial(
    pl.kernel,
    out_type=_F32L,
    mesh=_mesh,
    scratch_types=[
        pltpu.VMEM((NCHUNK, CHUNK), jnp.int32),
        pltpu.VMEM((NCHUNK, CHUNK), jnp.int32),
        pltpu.VMEM((EPW,), jnp.float32),
        pltpu.VMEM((EPW,), jnp.float32),
        pltpu.VMEM((EPW,), jnp.float32),
        pltpu.VMEM((EPW,), jnp.float32),
        pltpu.VMEM((EPW,), jnp.float32),
        pltpu.VMEM((EPW,), jnp.float32),
        pltpu.VMEM((EPW,), jnp.float32),
        pltpu.VMEM((EPW,), jnp.float32),
        pltpu.VMEM((EPW,), jnp.float32),
        pltpu.VMEM((EPW,), jnp.float32),
        pltpu.VMEM((8, LANES), jnp.float32),
        pltpu.VMEM((EPW,), jnp.float32),
        pltpu.SemaphoreType.DMA,
        pltpu.SemaphoreType.DMA,
    ],
)(_sc_b)


def kernel(idx1, idx2, weights, rs_loc, rs_scale, phis_loc, phis_scale,
           R_loc, R_scale, T, alpha_loc, alpha_scale):
    f32 = jnp.float32
    consts = jnp.stack([R_loc.astype(f32), R_scale.astype(f32),
                        T[0].astype(f32), T[1].astype(f32),
                        alpha_loc.astype(f32), alpha_scale.astype(f32),
                        f32(0.0), f32(0.0)])
    consts16 = jnp.broadcast_to(consts[:, None], (8, LANES))
    i1 = idx1.astype(jnp.int32)
    i2 = idx2.astype(jnp.int32)
    ph = phis_loc.astype(f32)
    cc, ss, rest = _sc_call_a(i1, i2, rs_loc.astype(f32),
                              rs_scale.astype(f32), phis_scale.astype(f32),
                              consts16)
    return _sc_call_b(i1, i2, weights.astype(f32), ph[:, 0], ph[:, 1],
                      ph[:, 2], cc, ss, rest, consts16)


# speedup vs baseline: 40.9634x; 1.0031x over previous
import functools

import jax
import jax.numpy as jnp
from jax import lax
from jax.experimental import pallas as pl
from jax.experimental.pallas import tpu as pltpu
from jax.experimental.pallas import tpu_sc as plsc

L_EDGES = 16384
NC = 2
NS = 16
NW = NC * NS
EPW = L_EDGES // NW
CHUNK = 128
NCHUNK = EPW // CHUNK
LANES = 16
VPC = CHUNK // LANES

_LN2 = 0.6931471805599453


def _ff(v):
    return jnp.full((LANES,), v, jnp.float32)


def _fi(v):
    return jnp.full((LANES,), v, jnp.int32)


def _vlog(x):
    xi = lax.bitcast_convert_type(x, jnp.int32)
    m = lax.bitcast_convert_type((xi & _fi(0x007FFFFF)) | _fi(0x3F800000),
                                 jnp.float32)
    e = (lax.shift_right_arithmetic(xi, _fi(23)) - _fi(127)).astype(jnp.float32)
    big = m > _ff(1.4142135)
    m = jnp.where(big, m * _ff(0.5), m)
    e = e + jnp.where(big, _ff(1.0), _ff(0.0))
    r = (m - _ff(1.0)) / (m + _ff(1.0))
    r2 = r * r
    t = ((_ff(1.0 / 7.0) * r2 + _ff(1.0 / 5.0)) * r2 + _ff(1.0 / 3.0)) * r2 + _ff(1.0)
    return e * _ff(_LN2) + _ff(2.0) * r * t


def _vsqrt(x):
    return jnp.exp(_ff(0.5) * _vlog(x))


def _sc_a(idx1, idx2, rs_loc, rs_scale, phis_scale, consts,
          cc_o, ss_o, rest_o, idx1_v, idx2_v, a1, a2, b1, b2, c1, c2,
          cv, cc_v, ss_v, rest_v, sem):
    wid = lax.axis_index("s") * NC + lax.axis_index("c")
    base = wid * EPW

    for j in range(NCHUNK):
        sl = pl.ds(base + j * CHUNK, CHUNK)
        pltpu.sync_copy(idx1.at[sl], idx1_v.at[j])
        pltpu.sync_copy(idx2.at[sl], idx2_v.at[j])
    pltpu.sync_copy(consts, cv)

    chunk_copies = []
    for j in range(NCHUNK):
        i1 = idx1_v.at[j]
        i2 = idx2_v.at[j]
        dsl = pl.ds(j * CHUNK, CHUNK)
        chunk_copies.append([
            pltpu.async_copy(rs_loc.at[i1], a1.at[dsl], sem),
            pltpu.async_copy(rs_loc.at[i2], a2.at[dsl], sem),
            pltpu.async_copy(rs_scale.at[i1], b1.at[dsl], sem),
            pltpu.async_copy(rs_scale.at[i2], b2.at[dsl], sem),
            pltpu.async_copy(phis_scale.at[i1], c1.at[dsl], sem),
            pltpu.async_copy(phis_scale.at[i2], c2.at[dsl], sem),
        ])

    eps = _ff(1e-12)
    one = _ff(1.0)
    half = _ff(0.5)
    Rv = jnp.exp(cv[0])
    av = jnp.exp(cv[4])
    ear = jnp.exp(av * Rv)
    lnv = _vlog(half * (ear + one / ear) - one + eps)
    er_s = jnp.exp(cv[1])
    ea_s = jnp.exp(cv[5])
    kl = half * (cv[0] * cv[0] + er_s * er_s) + half * (cv[4] * cv[4] + ea_s * ea_s)
    ctv = kl * _ff(1.0 / L_EDGES)

    def vec_body(k, _):
        sl = pl.ds(k * LANES, LANES)
        r1 = Rv / (one + jnp.exp(-a1[sl]))
        r2 = Rv / (one + jnp.exp(-a2[sl]))
        e1 = jnp.exp(r1)
        e2 = jnp.exp(r2)
        ch1 = half * (e1 + one / e1)
        sh1 = half * (e1 - one / e1)
        ch2 = half * (e2 + one / e2)
        sh2 = half * (e2 - one / e2)
        g1 = jnp.exp(av * r1)
        g2 = jnp.exp(av * r2)
        logr1 = _vlog(av * half * (g1 - one / g1) + eps) - lnv
        logr2 = _vlog(av * half * (g2 - one / g2) + eps) - lnv
        s12 = jnp.exp(b1[sl]) + jnp.exp(c1[sl]) + jnp.exp(b2[sl]) + jnp.exp(c2[sl])
        cc_v[sl] = ch1 * ch2
        ss_v[sl] = sh1 * sh2
        rest_v[sl] = logr1 + logr2 - _ff(1e-3) * s12 - ctv
        return 0

    for j in range(NCHUNK):
        for cp in chunk_copies[j]:
            cp.wait()
        lax.fori_loop(j * VPC, (j + 1) * VPC, vec_body, 0)

    osl = pl.ds(base, EPW)
    pltpu.sync_copy(cc_v, cc_o.at[osl])
    pltpu.sync_copy(ss_v, ss_o.at[osl])
    pltpu.sync_copy(rest_v, rest_o.at[osl])


def _sc_b(idx1, idx2, w, phx, phy, phz, cc, ss, rest, consts, out,
          idx1_v, idx2_v, w_v, cc_v, ss_v, rest_v,
          px1v, py1v, pz1v, px2v, py2v, pz2v, cv, ov, sem, sem2):
    wid = lax.axis_index("s") * NC + lax.axis_index("c")
    base = wid * EPW

    idx_cps = []
    for j in range(NCHUNK):
        sl = pl.ds(base + j * CHUNK, CHUNK)
        idx_cps.append(pltpu.async_copy(idx1.at[sl], idx1_v.at[j], sem2))
        idx_cps.append(pltpu.async_copy(idx2.at[sl], idx2_v.at[j], sem2))
    bsl = pl.ds(base, EPW)
    stage_cps = [
        pltpu.async_copy(w.at[bsl], w_v, sem2),
        pltpu.async_copy(cc.at[bsl], cc_v, sem2),
        pltpu.async_copy(ss.at[bsl], ss_v, sem2),
        pltpu.async_copy(rest.at[bsl], rest_v, sem2),
        pltpu.async_copy(consts, cv, sem2),
    ]
    for cp in idx_cps:
        cp.wait()

    chunk_copies = []
    for j in range(NCHUNK):
        i1 = idx1_v.at[j]
        i2 = idx2_v.at[j]
        dsl = pl.ds(j * CHUNK, CHUNK)
        chunk_copies.append([
            pltpu.async_copy(phx.at[i1], px1v.at[dsl], sem),
            pltpu.async_copy(phx.at[i2], px2v.at[dsl], sem),
            pltpu.async_copy(phy.at[i1], py1v.at[dsl], sem),
            pltpu.async_copy(phy.at[i2], py2v.at[dsl], sem),
            pltpu.async_copy(phz.at[i1], pz1v.at[dsl], sem),
            pltpu.async_copy(phz.at[i2], pz2v.at[dsl], sem),
        ])

    for cp in stage_cps:
        cp.wait()
    one = _ff(1.0)
    half = _ff(0.5)
    eps = _ff(1e-12)
    Rv = jnp.exp(cv[0])
    tx0 = jnp.exp(cv[2])
    tx1 = jnp.exp(cv[3])
    itv = one / (_ff(2.0) * (tx0 / (tx0 + tx1)) + eps)

    def vec_body(k, _):
        sl = pl.ds(k * LANES, LANES)
        px1 = px1v[sl]
        py1 = py1v[sl]
        pz1 = pz1v[sl]
        px2 = px2v[sl]
        py2 = py2v[sl]
        pz2 = pz2v[sl]
        n1 = px1 * px1 + py1 * py1 + pz1 * pz1
        n2 = px2 * px2 + py2 * py2 + pz2 * pz2
        dot = px1 * px2 + py1 * py2 + pz1 * pz2
        cos = dot * jnp.exp(-half * _vlog(n1 * n2))
        cos = jnp.minimum(jnp.maximum(cos, -one), one)
        ch = jnp.maximum(cc_v[sl] - ss_v[sl] * cos, _ff(1.0 + 1e-7))
        d = _vlog(ch + _vsqrt(ch * ch - one))
        z = (d - Rv) * itv
        sp = _vlog(one + jnp.exp(-jnp.abs(z)))
        lim = _ff(-27.631021)
        lp = jnp.maximum(-(jnp.maximum(z, _ff(0.0)) + sp), lim)
        l1mp = jnp.maximum(-(jnp.maximum(-z, _ff(0.0)) + sp), lim)
        llt = jnp.where(w_v[sl] > _ff(0.0), lp, l1mp)
        ov[sl] = llt + rest_v[sl]

    for j in range(NCHUNK):
        for cp in chunk_copies[j]:
            cp.wait()
        plsc.parallel_loop(j * VPC, (j + 1) * VPC, 1, unroll=4)(
            lambda k: vec_body(k, None))

    pltpu.sync_copy(ov, out.at[pl.ds(base, EPW)])


_mesh = plsc.VectorSubcoreMesh(core_axis_name="c", subcore_axis_name="s")

_F32L = jax.ShapeDtypeStruct((L_EDGES,), jnp.float32)

_sc_call_a = functools.partial(
    pl.kernel,
    out_type=(_F32L, _F32L, _F32L),
    mesh=_mesh,
    scratch_types=[
        pltpu.VMEM((NCHUNK, CHUNK), jnp.int32),
        pltpu.VMEM((NCHUNK, CHUNK), jnp.int32),
        pltpu.VMEM((EPW,), jnp.float32),
        pltpu.VMEM((EPW,), jnp.float32),
        pltpu.VMEM((EPW,), jnp.float32),
        pltpu.VMEM((EPW,), jnp.float32),
        pltpu.VMEM((EPW,), jnp.float32),
        pltpu.VMEM((EPW,), jnp.float32),
        pltpu.VMEM((8, LANES), jnp.float32),
        pltpu.VMEM((EPW,), jnp.float32),
        pltpu.VMEM((EPW,), jnp.float32),
        pltpu.VMEM((EPW,), jnp.float32),
        pltpu.SemaphoreType.DMA,
    ],
)(_sc_a)

_sc_call_b = functools.partial(
    pl.kernel,
    out_type=_F32L,
    mesh=_mesh,
    scratch_types=[
        pltpu.VMEM((NCHUNK, CHUNK), jnp.int32),
        pltpu.VMEM((NCHUNK, CHUNK), jnp.int32),
        pltpu.VMEM((EPW,), jnp.float32),
        pltpu.VMEM((EPW,), jnp.float32),
        pltpu.VMEM((EPW,), jnp.float32),
        pltpu.VMEM((EPW,), jnp.float32),
        pltpu.VMEM((EPW,), jnp.float32),
        pltpu.VMEM((EPW,), jnp.float32),
        pltpu.VMEM((EPW,), jnp.float32),
        pltpu.VMEM((EPW,), jnp.float32),
        pltpu.VMEM((EPW,), jnp.float32),
        pltpu.VMEM((EPW,), jnp.float32),
        pltpu.VMEM((8, LANES), jnp.float32),
        pltpu.VMEM((EPW,), jnp.float32),
        pltpu.SemaphoreType.DMA,
        pltpu.SemaphoreType.DMA,
    ],
)(_sc_b)


def kernel(idx1, idx2, weights, rs_loc, rs_scale, phis_loc, phis_scale,
           R_loc, R_scale, T, alpha_loc, alpha_scale):
    f32 = jnp.float32
    consts = jnp.stack([R_loc.astype(f32), R_scale.astype(f32),
                        T[0].astype(f32), T[1].astype(f32),
                        alpha_loc.astype(f32), alpha_scale.astype(f32),
                        f32(0.0), f32(0.0)])
    consts16 = jnp.broadcast_to(consts[:, None], (8, LANES))
    i1 = idx1.astype(jnp.int32)
    i2 = idx2.astype(jnp.int32)
    ph = phis_loc.astype(f32)
    cc, ss, rest = _sc_call_a(i1, i2, rs_loc.astype(f32),
                              rs_scale.astype(f32), phis_scale.astype(f32),
                              consts16)
    return _sc_call_b(i1, i2, weights.astype(f32), ph[:, 0], ph[:, 1],
                      ph[:, 2], cc, ss, rest, consts16)
